# baseline (speedup 1.0000x reference)
import jax
import jax.numpy as jnp
from jax import lax
from jax.experimental import pallas as pl
from jax.experimental.pallas import tpu as pltpu

CB = 1024
D = 32000
KB = 1280
NKB = D // KB
K = 5
TEMP = 0.1


def _fused_kernel(l_ref, ct_ref, idx_ref, dist_ref, q_ref, acc_ref, wsp_ref):
    s = pl.program_id(0)
    ct_blk = ct_ref[...]

    @pl.when(s < NKB)
    def _():
        l_blk = l_ref[...]
        dot = lax.dot_general(l_blk, ct_blk, (((1,), (0,)), ((), ())),
                              preferred_element_type=jnp.float32)
        c2 = jnp.sum(ct_blk * ct_blk, axis=0, keepdims=True)
        l2 = jnp.sum(l_blk * l_blk, axis=1, keepdims=True)
        part = l2 + c2 - 2.0 * dot

        @pl.when(s == 0)
        def _():
            acc_ref[...] = part

        @pl.when(s > 0)
        def _():
            acc_ref[...] = acc_ref[...] + part

        @pl.when(s == NKB - 1)
        def _():
            d2 = acc_ref[...]
            lane = lax.broadcasted_iota(jnp.int32, (16, CB), 1)
            out_lane = lax.broadcasted_iota(jnp.int32, (16, 128), 1)
            idx_acc = jnp.zeros((16, 128), jnp.int32)
            d2_acc = jnp.zeros((16, 128), jnp.float32)
            cur = d2
            mis = []
            for k in range(K):
                mv = jnp.min(cur, axis=1, keepdims=True)
                cand = jnp.where(cur == mv, lane, CB)
                mi = jnp.min(cand, axis=1, keepdims=True)
                mis.append(mi)
                idx_acc = jnp.where(out_lane == k, mi, idx_acc)
                d2_acc = jnp.where(out_lane == k, mv, d2_acc)
                cur = jnp.where(lane == mi, jnp.float32(3.0e38), cur)
            dist = jnp.sqrt(jnp.maximum(d2_acc, 1e-12))
            valid = out_lane < K
            logits = -dist / TEMP
            m = jnp.max(jnp.where(valid, logits, -3.0e38), axis=1,
                        keepdims=True)
            e = jnp.where(valid, jnp.exp(logits - m), 0.0)
            w = e / jnp.sum(e, axis=1, keepdims=True)
            wsp = jnp.zeros((16, CB), jnp.float32)
            for k in range(K):
                wk = jnp.sum(jnp.where(out_lane == k, w, 0.0), axis=1,
                             keepdims=True)
                wsp = jnp.where(lane == mis[k], wk, wsp)
            idx_ref[...] = idx_acc
            dist_ref[...] = dist
            wsp_ref[...] = wsp

    @pl.when(s >= NKB)
    def _():
        q_ref[...] = lax.dot_general(
            wsp_ref[...], ct_blk, (((1,), (1,)), ((), ())),
            preferred_element_type=jnp.float32)


def _soft_quantize(latent_flat, ct):
    out_shapes = (
        jax.ShapeDtypeStruct((16, 128), jnp.int32),
        jax.ShapeDtypeStruct((16, 128), jnp.float32),
        jax.ShapeDtypeStruct((16, D), jnp.float32),
    )
    return pl.pallas_call(
        _fused_kernel,
        grid=(2 * NKB,),
        in_specs=[
            pl.BlockSpec((16, KB), lambda s: (0, s % NKB)),
            pl.BlockSpec((KB, CB), lambda s: (s % NKB, 0)),
        ],
        out_specs=(
            pl.BlockSpec((16, 128), lambda s: (0, 0)),
            pl.BlockSpec((16, 128), lambda s: (0, 0)),
            pl.BlockSpec((16, KB), lambda s: (0, s % NKB)),
        ),
        out_shape=out_shapes,
        scratch_shapes=[
            pltpu.VMEM((16, CB), jnp.float32),
            pltpu.VMEM((16, CB), jnp.float32),
        ],
        compiler_params=pltpu.CompilerParams(
            dimension_semantics=("arbitrary",),
        ),
    )(latent_flat, ct)


def kernel(latent, codebook):
    B = latent.shape[0]
    latent_flat = latent.reshape(B, -1).astype(jnp.float32)
    ct = codebook.transpose(1, 2, 3, 0).reshape(D, CB).astype(jnp.float32)

    idx_pad, dist_pad, quantized_flat = _soft_quantize(latent_flat, ct)
    main_indices = idx_pad[:, 0]
    main_distances = dist_pad[:, 0]
    quantized = quantized_flat.reshape(latent.shape).astype(latent.dtype)
    return quantized, main_indices, main_distances


# speedup vs baseline: 1.0447x; 1.0447x over previous
import jax
import jax.numpy as jnp
from jax import lax
from jax.experimental import pallas as pl
from jax.experimental.pallas import tpu as pltpu

CB = 1024
D = 32000
KB = 6400
NKB = D // KB
K = 5
TEMP = 0.1


def _dist_kernel(l_ref, ct_ref, idx_ref, w_ref, dist_ref, wsp_ref, acc_ref):
    k_step = pl.program_id(0)
    l_blk = l_ref[...]
    ct_blk = ct_ref[...]
    dot = lax.dot_general(l_blk, ct_blk, (((1,), (0,)), ((), ())),
                          preferred_element_type=jnp.float32)
    c2 = jnp.sum(ct_blk * ct_blk, axis=0, keepdims=True)
    l2 = jnp.sum(l_blk * l_blk, axis=1, keepdims=True)
    part = l2 + c2 - 2.0 * dot

    @pl.when(k_step == 0)
    def _():
        acc_ref[...] = part

    @pl.when(k_step > 0)
    def _():
        acc_ref[...] = acc_ref[...] + part

    @pl.when(k_step == NKB - 1)
    def _():
        d2 = acc_ref[...]
        lane = lax.broadcasted_iota(jnp.int32, (16, CB), 1)
        out_lane = lax.broadcasted_iota(jnp.int32, (16, 128), 1)
        idx_acc = jnp.zeros((16, 128), jnp.int32)
        d2_acc = jnp.zeros((16, 128), jnp.float32)
        cur = d2
        mis = []
        for k in range(K):
            mv = jnp.min(cur, axis=1, keepdims=True)
            cand = jnp.where(cur == mv, lane, CB)
            mi = jnp.min(cand, axis=1, keepdims=True)
            mis.append(mi)
            idx_acc = jnp.where(out_lane == k, mi, idx_acc)
            d2_acc = jnp.where(out_lane == k, mv, d2_acc)
            cur = jnp.where(lane == mi, jnp.float32(3.0e38), cur)
        dist = jnp.sqrt(jnp.maximum(d2_acc, 1e-12))
        valid = out_lane < K
        logits = -dist / TEMP
        m = jnp.max(jnp.where(valid, logits, -3.0e38), axis=1, keepdims=True)
        e = jnp.where(valid, jnp.exp(logits - m), 0.0)
        w = e / jnp.sum(e, axis=1, keepdims=True)
        wsp = jnp.zeros((16, CB), jnp.float32)
        for k in range(K):
            wk = jnp.sum(jnp.where(out_lane == k, w, 0.0), axis=1,
                         keepdims=True)
            wsp = jnp.where(lane == mis[k], wk, wsp)
        idx_ref[...] = idx_acc
        w_ref[...] = w
        dist_ref[...] = dist
        wsp_ref[...] = wsp


def _distances_top5(latent_flat, ct):
    out_shapes = (
        jax.ShapeDtypeStruct((16, 128), jnp.int32),
        jax.ShapeDtypeStruct((16, 128), jnp.float32),
        jax.ShapeDtypeStruct((16, 128), jnp.float32),
        jax.ShapeDtypeStruct((16, CB), jnp.float32),
    )
    return pl.pallas_call(
        _dist_kernel,
        grid=(NKB,),
        in_specs=[
            pl.BlockSpec((16, KB), lambda k: (0, k)),
            pl.BlockSpec((KB, CB), lambda k: (k, 0)),
        ],
        out_specs=(
            pl.BlockSpec((16, 128), lambda k: (0, 0)),
            pl.BlockSpec((16, 128), lambda k: (0, 0)),
            pl.BlockSpec((16, 128), lambda k: (0, 0)),
            pl.BlockSpec((16, CB), lambda k: (0, 0)),
        ),
        out_shape=out_shapes,
        scratch_shapes=[pltpu.VMEM((16, CB), jnp.float32)],
        compiler_params=pltpu.CompilerParams(
            dimension_semantics=("arbitrary",),
        ),
    )(latent_flat, ct)


def _wsum_kernel(wsp_ref, ct_ref, out_ref):
    out_ref[...] = lax.dot_general(
        wsp_ref[...], ct_ref[...], (((1,), (1,)), ((), ())),
        preferred_element_type=jnp.float32)


def _weighted_sum(wsp, ct):
    return pl.pallas_call(
        _wsum_kernel,
        grid=(NKB,),
        in_specs=[
            pl.BlockSpec((16, CB), lambda k: (0, 0)),
            pl.BlockSpec((KB, CB), lambda k: (k, 0)),
        ],
        out_specs=pl.BlockSpec((16, KB), lambda k: (0, k)),
        out_shape=jax.ShapeDtypeStruct((16, D), jnp.float32),
        compiler_params=pltpu.CompilerParams(
            dimension_semantics=("arbitrary",),
        ),
    )(wsp, ct)


def kernel(latent, codebook):
    B = latent.shape[0]
    latent_flat = latent.reshape(B, -1).astype(jnp.float32)
    ct = codebook.transpose(1, 2, 3, 0).reshape(D, CB).astype(jnp.float32)

    idx_pad, w_pad, dist_pad, wsp = _distances_top5(latent_flat, ct)
    main_indices = idx_pad[:, 0]
    main_distances = dist_pad[:, 0]

    quantized_flat = _weighted_sum(wsp, ct)
    quantized = quantized_flat.reshape(latent.shape).astype(latent.dtype)
    return quantized, main_indices, main_distances


# speedup vs baseline: 1.0834x; 1.0371x over previous
import jax
import jax.numpy as jnp
from jax import lax
from jax.experimental import pallas as pl
from jax.experimental.pallas import tpu as pltpu

CB = 1024
D = 32000
KB = 3200
NKB = D // KB
K = 5
TEMP = 0.1


def _dist_kernel(l_ref, ct_ref, idx_ref, w_ref, dist_ref, wsp_ref, acc_ref):
    k_step = pl.program_id(0)
    l_blk = l_ref[...]
    ct_blk = ct_ref[...]
    dot = lax.dot_general(l_blk, ct_blk, (((1,), (0,)), ((), ())),
                          preferred_element_type=jnp.float32)
    c2 = jnp.sum(ct_blk * ct_blk, axis=0, keepdims=True)
    l2 = jnp.sum(l_blk * l_blk, axis=1, keepdims=True)
    part = l2 + c2 - 2.0 * dot

    @pl.when(k_step == 0)
    def _():
        acc_ref[...] = part

    @pl.when(k_step > 0)
    def _():
        acc_ref[...] = acc_ref[...] + part

    @pl.when(k_step == NKB - 1)
    def _():
        d2 = acc_ref[...]
        lane = lax.broadcasted_iota(jnp.int32, (16, CB), 1)
        out_lane = lax.broadcasted_iota(jnp.int32, (16, 128), 1)
        idx_acc = jnp.zeros((16, 128), jnp.int32)
        d2_acc = jnp.zeros((16, 128), jnp.float32)
        cur = d2
        mis = []
        for k in range(K):
            mv = jnp.min(cur, axis=1, keepdims=True)
            cand = jnp.where(cur == mv, lane, CB)
            mi = jnp.min(cand, axis=1, keepdims=True)
            mis.append(mi)
            idx_acc = jnp.where(out_lane == k, mi, idx_acc)
            d2_acc = jnp.where(out_lane == k, mv, d2_acc)
            cur = jnp.where(lane == mi, jnp.float32(3.0e38), cur)
        dist = jnp.sqrt(jnp.maximum(d2_acc, 1e-12))
        valid = out_lane < K
        logits = -dist / TEMP
        m = jnp.max(jnp.where(valid, logits, -3.0e38), axis=1, keepdims=True)
        e = jnp.where(valid, jnp.exp(logits - m), 0.0)
        w = e / jnp.sum(e, axis=1, keepdims=True)
        wsp = jnp.zeros((16, CB), jnp.float32)
        for k in range(K):
            wk = jnp.sum(jnp.where(out_lane == k, w, 0.0), axis=1,
                         keepdims=True)
            wsp = jnp.where(lane == mis[k], wk, wsp)
        idx_ref[...] = idx_acc
        w_ref[...] = w
        dist_ref[...] = dist
        wsp_ref[...] = wsp


def _distances_top5(latent_flat, ct):
    out_shapes = (
        jax.ShapeDtypeStruct((16, 128), jnp.int32),
        jax.ShapeDtypeStruct((16, 128), jnp.float32),
        jax.ShapeDtypeStruct((16, 128), jnp.float32),
        jax.ShapeDtypeStruct((16, CB), jnp.float32),
    )
    return pl.pallas_call(
        _dist_kernel,
        grid=(NKB,),
        in_specs=[
            pl.BlockSpec((16, KB), lambda k: (0, k)),
            pl.BlockSpec((KB, CB), lambda k: (k, 0)),
        ],
        out_specs=(
            pl.BlockSpec((16, 128), lambda k: (0, 0)),
            pl.BlockSpec((16, 128), lambda k: (0, 0)),
            pl.BlockSpec((16, 128), lambda k: (0, 0)),
            pl.BlockSpec((16, CB), lambda k: (0, 0)),
        ),
        out_shape=out_shapes,
        scratch_shapes=[pltpu.VMEM((16, CB), jnp.float32)],
        compiler_params=pltpu.CompilerParams(
            dimension_semantics=("arbitrary",),
        ),
    )(latent_flat, ct)


def _wsum_kernel(wsp_ref, ct_ref, out_ref):
    out_ref[...] = lax.dot_general(
        wsp_ref[...], ct_ref[...], (((1,), (1,)), ((), ())),
        preferred_element_type=jnp.float32)


def _weighted_sum(wsp, ct):
    return pl.pallas_call(
        _wsum_kernel,
        grid=(NKB,),
        in_specs=[
            pl.BlockSpec((16, CB), lambda k: (0, 0)),
            pl.BlockSpec((KB, CB), lambda k: (k, 0)),
        ],
        out_specs=pl.BlockSpec((16, KB), lambda k: (0, k)),
        out_shape=jax.ShapeDtypeStruct((16, D), jnp.float32),
        compiler_params=pltpu.CompilerParams(
            dimension_semantics=("arbitrary",),
        ),
    )(wsp, ct)


def kernel(latent, codebook):
    B = latent.shape[0]
    latent_flat = latent.reshape(B, -1).astype(jnp.float32)
    ct = codebook.transpose(1, 2, 3, 0).reshape(D, CB).astype(jnp.float32)

    idx_pad, w_pad, dist_pad, wsp = _distances_top5(latent_flat, ct)
    main_indices = idx_pad[:, 0]
    main_distances = dist_pad[:, 0]

    quantized_flat = _weighted_sum(wsp, ct)
    quantized = quantized_flat.reshape(latent.shape).astype(latent.dtype)
    return quantized, main_indices, main_distances
